# Initial kernel scaffold; baseline (speedup 1.0000x reference)
import jax
import jax.numpy as jnp
from jax.experimental import pallas as pl


def kernel(z, u, x, edge_index, batch, batch_size, prev_h, W_glob, b_glob, W_ih, W_hh, b_ih, b_hh):
    raise NotImplementedError("write your pallas kernel here")


# speedup vs baseline: 2.9276x; 2.9276x over previous
import jax
import jax.numpy as jnp
from jax.experimental import pallas as pl
from jax.experimental.pallas import tpu as pltpu


def _fused_kernel(bs_res_ref, batch_ref, z_ref, u_ref, prev_h_ref,
                  W_glob_ref, b_glob_ref, W_ih_ref, W_hh_ref, b_ih_ref,
                  b_hh_ref, out_ref, acc_ref):
    i = pl.program_id(0)
    k = pl.num_programs(0)

    @pl.when(i == 0)
    def _init():
        acc_ref[...] = jnp.zeros_like(acc_ref)

    zb = z_ref[...]
    bb = batch_ref[...]
    R = zb.shape[0]
    num_graphs = acc_ref.shape[0]
    seg_iota = jax.lax.broadcasted_iota(jnp.int32, (R, num_graphs), 1)
    onehot = (bb == seg_iota).astype(jnp.float32)
    zb_aug = jnp.concatenate([zb, jnp.ones((R, 8), jnp.float32)], axis=1)
    part = jax.lax.dot_general(onehot, zb_aug,
                               (((0,), (0,)), ((), ())),
                               preferred_element_type=jnp.float32)
    acc_ref[...] += part

    @pl.when(i == k - 1)
    def _finish():
        acc = acc_ref[...]
        seg_sum = acc[:, :128]
        counts = acc[:, 128:129]
        graph_emb = seg_sum / jnp.maximum(counts, 1.0) + bs_res_ref[0, 0]
        u = u_ref[...]
        glob = jax.lax.dot_general(u, W_glob_ref[...],
                                   (((1,), (1,)), ((), ())),
                                   preferred_element_type=jnp.float32)
        glob = jnp.maximum(glob + b_glob_ref[...], 0.0)
        fused = jnp.concatenate([graph_emb, glob], axis=1)
        gi = jax.lax.dot_general(fused, W_ih_ref[...],
                                 (((1,), (1,)), ((), ())),
                                 preferred_element_type=jnp.float32) + b_ih_ref[...]
        prev_h = prev_h_ref[...]
        gh = jax.lax.dot_general(prev_h, W_hh_ref[...],
                                 (((1,), (1,)), ((), ())),
                                 preferred_element_type=jnp.float32) + b_hh_ref[...]
        d_h = prev_h.shape[1]
        i_r, i_z, i_n = gi[:, :d_h], gi[:, d_h:2 * d_h], gi[:, 2 * d_h:]
        h_r, h_z, h_n = gh[:, :d_h], gh[:, d_h:2 * d_h], gh[:, 2 * d_h:]
        r = jax.nn.sigmoid(i_r + h_r)
        zg = jax.nn.sigmoid(i_z + h_z)
        n = jnp.tanh(i_n + r * h_n)
        out_ref[...] = (1.0 - zg) * n + zg * prev_h


def kernel(z, u, x, edge_index, batch, batch_size, prev_h, W_glob, b_glob,
           W_ih, W_hh, b_ih, b_hh):
    del x, edge_index
    N, d_z = z.shape
    G, d_h = prev_h.shape
    d_u = u.shape[1]
    bs_res = (jnp.asarray(batch_size, jnp.float32) - G).reshape(1, 1)

    R = 1000 if N % 1000 == 0 else N
    K = N // R
    batch2d = batch.reshape(N, 1)

    full = lambda i: (0, 0)
    out = pl.pallas_call(
        _fused_kernel,
        grid=(K,),
        in_specs=[
            pl.BlockSpec((1, 1), full),
            pl.BlockSpec((R, 1), lambda i: (i, 0)),
            pl.BlockSpec((R, d_z), lambda i: (i, 0)),
            pl.BlockSpec((G, d_u), full),
            pl.BlockSpec((G, d_h), full),
            pl.BlockSpec(W_glob.shape, full),
            pl.BlockSpec((1, b_glob.shape[0]), full),
            pl.BlockSpec(W_ih.shape, full),
            pl.BlockSpec(W_hh.shape, full),
            pl.BlockSpec((1, b_ih.shape[0]), full),
            pl.BlockSpec((1, b_hh.shape[0]), full),
        ],
        out_specs=pl.BlockSpec((G, d_h), full),
        out_shape=jax.ShapeDtypeStruct((G, d_h), jnp.float32),
        scratch_shapes=[pltpu.VMEM((G, d_z + 8), jnp.float32)],
    )(bs_res, batch2d, z, u, prev_h, W_glob, b_glob.reshape(1, -1),
      W_ih, W_hh, b_ih.reshape(1, -1), b_hh.reshape(1, -1))
    return (out, out)


# speedup vs baseline: 4.3466x; 1.4847x over previous
import functools

import jax
import jax.numpy as jnp
from jax import lax
from jax.experimental import pallas as pl
from jax.experimental.pallas import tpu as pltpu
from jax.experimental.pallas import tpu_sc as plsc

_C = 128
_CW = 128


def _sc_segsum_body(nfull, ntail, niter,
                    z, batch, zzero, czero, ones_in,
                    out, outc,
                    acc, cnt, zbuf, idxbuf, onesbuf, ztail, idxtail, onestail):
    cidx = lax.axis_index("c")
    sid = lax.axis_index("s")
    wid = sid * 2 + cidx

    @pl.when(sid == 0)
    def _init():
        pltpu.sync_copy(zzero, acc)
        pltpu.sync_copy(czero, cnt)

    pltpu.sync_copy(ones_in, onesbuf)
    if ntail:
        pltpu.sync_copy(ones_in.at[pl.ds(0, ntail)], onestail)
    plsc.subcore_barrier()

    def step(i, carry):
        c = wid + 32 * i

        @pl.when(c < nfull)
        def _full():
            base = c * _C
            pltpu.sync_copy(z.at[pl.ds(base, _C)], zbuf)
            pltpu.sync_copy(batch.at[pl.ds(base, _C)], idxbuf)
            pltpu.sync_copy(zbuf, acc.at[idxbuf], add=True)
            pltpu.sync_copy(onesbuf, cnt.at[idxbuf], add=True)

        if ntail:
            @pl.when(c == nfull)
            def _tail():
                base = nfull * _C
                pltpu.sync_copy(z.at[pl.ds(base, ntail)], ztail)
                pltpu.sync_copy(batch.at[pl.ds(base, ntail)], idxtail)
                pltpu.sync_copy(ztail, acc.at[idxtail], add=True)
                pltpu.sync_copy(onestail, cnt.at[idxtail], add=True)
        return carry

    lax.fori_loop(0, niter, step, None)
    plsc.subcore_barrier()

    @pl.when(sid == 0)
    def _flush():
        pltpu.sync_copy(acc, out.at[cidx])
        pltpu.sync_copy(cnt, outc.at[cidx])


def _sc_segment_sum(z, batch, num_graphs):
    N, d_z = z.shape
    nfull = N // _C
    ntail = N - nfull * _C
    nchunks = nfull + (1 if ntail else 0)
    niter = (nchunks + 31) // 32

    zzero = jnp.zeros((num_graphs, d_z), jnp.float32)
    czero = jnp.zeros((num_graphs, _CW), jnp.float32)
    ones_in = jnp.ones((_C, _CW), jnp.float32)

    mesh = plsc.VectorSubcoreMesh(core_axis_name="c", subcore_axis_name="s")
    body = functools.partial(_sc_segsum_body, nfull, ntail, niter)
    scratch = [
        pltpu.VMEM_SHARED((num_graphs, d_z), jnp.float32),
        pltpu.VMEM_SHARED((num_graphs, _CW), jnp.float32),
        pltpu.VMEM((_C, d_z), jnp.float32),
        pltpu.VMEM((_C,), jnp.int32),
        pltpu.VMEM((_C, _CW), jnp.float32),
        pltpu.VMEM((max(ntail, 1), d_z), jnp.float32),
        pltpu.VMEM((max(ntail, 1),), jnp.int32),
        pltpu.VMEM((max(ntail, 1), _CW), jnp.float32),
    ]
    out_type = (jax.ShapeDtypeStruct((2, num_graphs, d_z), jnp.float32),
                jax.ShapeDtypeStruct((2, num_graphs, _CW), jnp.float32))
    return pl.kernel(body, out_type, mesh=mesh, scratch_types=scratch)(
        z, batch, zzero, czero, ones_in)


def _dense_body(bs_ref, p_ref, c_ref, u_ref, ph_ref, Wg_ref, bg_ref,
                Wih_ref, Whh_ref, bih_ref, bhh_ref, out_ref):
    seg = p_ref[0] + p_ref[1]
    counts = (c_ref[0] + c_ref[1])[:, :1]
    graph_emb = seg / jnp.maximum(counts, 1.0) + bs_ref[0, 0]
    glob = jax.lax.dot_general(u_ref[...], Wg_ref[...], (((1,), (1,)), ((), ())),
                               preferred_element_type=jnp.float32)
    glob = jnp.maximum(glob + bg_ref[...], 0.0)
    fused = jnp.concatenate([graph_emb, glob], axis=1)
    gi = jax.lax.dot_general(fused, Wih_ref[...], (((1,), (1,)), ((), ())),
                             preferred_element_type=jnp.float32) + bih_ref[...]
    ph = ph_ref[...]
    gh = jax.lax.dot_general(ph, Whh_ref[...], (((1,), (1,)), ((), ())),
                             preferred_element_type=jnp.float32) + bhh_ref[...]
    d_h = ph.shape[1]
    i_r, i_z, i_n = gi[:, :d_h], gi[:, d_h:2 * d_h], gi[:, 2 * d_h:]
    h_r, h_z, h_n = gh[:, :d_h], gh[:, d_h:2 * d_h], gh[:, 2 * d_h:]
    r = jax.nn.sigmoid(i_r + h_r)
    zg = jax.nn.sigmoid(i_z + h_z)
    n = jnp.tanh(i_n + r * h_n)
    out_ref[...] = (1.0 - zg) * n + zg * ph


def kernel(z, u, x, edge_index, batch, batch_size, prev_h, W_glob, b_glob,
           W_ih, W_hh, b_ih, b_hh):
    del x, edge_index
    G, d_h = prev_h.shape
    bs_res = (jnp.asarray(batch_size, jnp.float32) - G).reshape(1, 1)

    partials, cnts = _sc_segment_sum(z, batch, G)

    out = pl.pallas_call(
        _dense_body,
        out_shape=jax.ShapeDtypeStruct((G, d_h), jnp.float32),
    )(bs_res, partials, cnts, u, prev_h, W_glob, b_glob.reshape(1, -1),
      W_ih, W_hh, b_ih.reshape(1, -1), b_hh.reshape(1, -1))
    return (out, out)


# speedup vs baseline: 5.9471x; 1.3682x over previous
import functools

import jax
import jax.numpy as jnp
from jax import lax
from jax.experimental import pallas as pl
from jax.experimental.pallas import tpu as pltpu
from jax.experimental.pallas import tpu_sc as plsc

_C = 128
_CW = 128


def _sc_segsum_body(nfull, ntail, niter,
                    z, batch, zzero, czero, ones_in,
                    out, outc,
                    acc, cnt, zbuf0, zbuf1, idx2, onesbuf,
                    ztail, idxtail, onestail, semz0, semz1, semi0, semi1):
    cidx = lax.axis_index("c")
    sid = lax.axis_index("s")
    wid = sid * 2 + cidx
    zbufs = (zbuf0, zbuf1)
    semz = (semz0, semz1)
    semi = (semi0, semi1)

    @pl.when(sid == 0)
    def _init():
        pltpu.sync_copy(zzero, acc)
        pltpu.sync_copy(czero, cnt)

    pltpu.sync_copy(ones_in, onesbuf)
    if ntail:
        pltpu.sync_copy(ones_in.at[pl.ds(0, ntail)], onestail)

    def fire(c, b):
        @pl.when(c < nfull)
        def _():
            base = c * _C
            pltpu.async_copy(z.at[pl.ds(base, _C)], zbufs[b], semz[b])
            pltpu.async_copy(batch.at[pl.ds(base, _C)], idx2.at[b], semi[b])

    def drain_and_scatter(c, b):
        @pl.when(c < nfull)
        def _():
            base = c * _C
            pltpu.make_async_copy(z.at[pl.ds(base, _C)], zbufs[b], semz[b]).wait()
            pltpu.make_async_copy(batch.at[pl.ds(base, _C)], idx2.at[b], semi[b]).wait()
            pltpu.sync_copy(zbufs[b], acc.at[idx2.at[b]], add=True)
            pltpu.sync_copy(onesbuf, cnt.at[idx2.at[b]], add=True)

        if ntail:
            @pl.when(c == nfull)
            def _tail():
                base = nfull * _C
                pltpu.sync_copy(z.at[pl.ds(base, ntail)], ztail)
                pltpu.sync_copy(batch.at[pl.ds(base, ntail)], idxtail)
                pltpu.sync_copy(ztail, acc.at[idxtail], add=True)
                pltpu.sync_copy(onestail, cnt.at[idxtail], add=True)

    fire(wid, 0)
    plsc.subcore_barrier()

    npairs = (niter + 1) // 2

    def step(p, carry):
        i0 = 2 * p
        for b in (0, 1):
            i = i0 + b
            c = wid + 32 * i
            fire(wid + 32 * (i + 1), 1 - b)
            drain_and_scatter(c, b)
        return carry

    lax.fori_loop(0, npairs, step, None)
    plsc.subcore_barrier()

    @pl.when(sid == 0)
    def _flush():
        pltpu.sync_copy(acc, out.at[cidx])
        pltpu.sync_copy(cnt, outc.at[cidx])


def _sc_segment_sum(z, batch, num_graphs):
    N, d_z = z.shape
    nfull = N // _C
    ntail = N - nfull * _C
    nchunks = nfull + (1 if ntail else 0)
    niter = (nchunks + 31) // 32

    zzero = jnp.zeros((num_graphs, d_z), jnp.float32)
    czero = jnp.zeros((num_graphs, _CW), jnp.float32)
    ones_in = jnp.ones((_C, _CW), jnp.float32)

    mesh = plsc.VectorSubcoreMesh(core_axis_name="c", subcore_axis_name="s")
    body = functools.partial(_sc_segsum_body, nfull, ntail, niter)
    scratch = [
        pltpu.VMEM_SHARED((num_graphs, d_z), jnp.float32),
        pltpu.VMEM_SHARED((num_graphs, _CW), jnp.float32),
        pltpu.VMEM((_C, d_z), jnp.float32),
        pltpu.VMEM((_C, d_z), jnp.float32),
        pltpu.VMEM((2, _C), jnp.int32),
        pltpu.VMEM((_C, _CW), jnp.float32),
        pltpu.VMEM((max(ntail, 1), d_z), jnp.float32),
        pltpu.VMEM((max(ntail, 1),), jnp.int32),
        pltpu.VMEM((max(ntail, 1), _CW), jnp.float32),
        pltpu.SemaphoreType.DMA,
        pltpu.SemaphoreType.DMA,
        pltpu.SemaphoreType.DMA,
        pltpu.SemaphoreType.DMA,
    ]
    out_type = (jax.ShapeDtypeStruct((2, num_graphs, d_z), jnp.float32),
                jax.ShapeDtypeStruct((2, num_graphs, _CW), jnp.float32))
    return pl.kernel(body, out_type, mesh=mesh, scratch_types=scratch)(
        z, batch, zzero, czero, ones_in)


def _dense_body(bs_ref, p_ref, c_ref, u_ref, ph_ref, Wg_ref, bg_ref,
                Wih_ref, Whh_ref, bih_ref, bhh_ref, out_ref):
    seg = p_ref[0] + p_ref[1]
    counts = (c_ref[0] + c_ref[1])[:, :1]
    graph_emb = seg / jnp.maximum(counts, 1.0) + bs_ref[0, 0]
    glob = jax.lax.dot_general(u_ref[...], Wg_ref[...], (((1,), (1,)), ((), ())),
                               preferred_element_type=jnp.float32)
    glob = jnp.maximum(glob + bg_ref[...], 0.0)
    fused = jnp.concatenate([graph_emb, glob], axis=1)
    gi = jax.lax.dot_general(fused, Wih_ref[...], (((1,), (1,)), ((), ())),
                             preferred_element_type=jnp.float32) + bih_ref[...]
    ph = ph_ref[...]
    gh = jax.lax.dot_general(ph, Whh_ref[...], (((1,), (1,)), ((), ())),
                             preferred_element_type=jnp.float32) + bhh_ref[...]
    d_h = ph.shape[1]
    i_r, i_z, i_n = gi[:, :d_h], gi[:, d_h:2 * d_h], gi[:, 2 * d_h:]
    h_r, h_z, h_n = gh[:, :d_h], gh[:, d_h:2 * d_h], gh[:, 2 * d_h:]
    r = jax.nn.sigmoid(i_r + h_r)
    zg = jax.nn.sigmoid(i_z + h_z)
    n = jnp.tanh(i_n + r * h_n)
    out_ref[...] = (1.0 - zg) * n + zg * ph


def kernel(z, u, x, edge_index, batch, batch_size, prev_h, W_glob, b_glob,
           W_ih, W_hh, b_ih, b_hh):
    del x, edge_index
    G, d_h = prev_h.shape
    bs_res = (jnp.asarray(batch_size, jnp.float32) - G).reshape(1, 1)

    partials, cnts = _sc_segment_sum(z, batch, G)

    out = pl.pallas_call(
        _dense_body,
        out_shape=jax.ShapeDtypeStruct((G, d_h), jnp.float32),
    )(bs_res, partials, cnts, u, prev_h, W_glob, b_glob.reshape(1, -1),
      W_ih, W_hh, b_ih.reshape(1, -1), b_hh.reshape(1, -1))
    return (out, out)
